# baseline (speedup 1.0000x reference)
import functools

import jax
import jax.numpy as jnp
from jax import lax
from jax.experimental import pallas as pl
from jax.experimental.pallas import tpu as pltpu
from jax.experimental.pallas import tpu_sc as plsc

D_FEAT = 128
D_EDGE = 16
D_TIME = 100
D_MODEL = 128
L_SLOTS = 40
BP = 10240
ROWS = BP * L_SLOTS
NW = 32
RPW = ROWS // NW
CH = 128
NCH = RPW // CH
SRC_RPW = BP // NW
SCH = 64
SNCH = SRC_RPW // SCH

BQ = 256
J0 = 32
JLO = D_TIME - J0
P_DEG = 12
P_PAD = 16

_INV2PI = 0.15915494309189535
_PI2_HI = 6.28125
_PI2_LO = 0.0019353071795864769


def _fast_cos(x):
    f32 = jnp.float32
    xa = jnp.abs(x)
    kf = (xa * f32(_INV2PI) + f32(0.5)).astype(jnp.int32).astype(f32)
    r = (xa - kf * f32(_PI2_HI)) - kf * f32(_PI2_LO)
    r2 = r * r
    acc = f32(4.7794773e-14)
    for c in (-1.1470746e-11, 2.0876757e-09, -2.7557319e-07,
              2.4801587e-05, -1.3888889e-03, 4.1666668e-02, -0.5):
        acc = acc * r2 + f32(c)
    return acc * r2 + f32(1.0)


def _sc_gather(node_tab, edge_tab, nbr_idx, eidx, sidx):
    mesh = plsc.VectorSubcoreMesh(core_axis_name="c", subcore_axis_name="s")

    @functools.partial(
        pl.kernel,
        out_type=[
            jax.ShapeDtypeStruct((ROWS, D_FEAT), jnp.bfloat16),
            jax.ShapeDtypeStruct((ROWS, D_EDGE), jnp.float32),
            jax.ShapeDtypeStruct((BP, D_FEAT), jnp.bfloat16),
        ],
        mesh=mesh,
        compiler_params=pltpu.CompilerParams(use_tc_tiling_on_sc=False),
        scratch_types=[
            pltpu.VMEM((RPW,), jnp.int32),
            pltpu.VMEM((RPW,), jnp.int32),
            pltpu.VMEM((4, CH, D_FEAT), jnp.bfloat16),
            pltpu.VMEM((4, CH, D_EDGE), jnp.float32),
            pltpu.VMEM((SCH, D_FEAT), jnp.bfloat16),
            pltpu.SemaphoreType.DMA,
        ] + [pltpu.SemaphoreType.DMA] * 12,
    )
    def k(node_hbm, edge_hbm, nidx_hbm, eidx_hbm, sidx_hbm,
          ng_out, eg_out, src_out,
          nidx_v, eidx_v, nbuf, ebuf, sbuf, nsem, *sems):
        gsem = sems[0:4]
        esem = sems[4:8]
        wsem = sems[8:12]
        wid = lax.axis_index("s") * 2 + lax.axis_index("c")
        base = wid * RPW
        pltpu.sync_copy(nidx_hbm.at[pl.ds(base, RPW)], nidx_v)
        pltpu.sync_copy(eidx_hbm.at[pl.ds(base, RPW)], eidx_v)

        def g_descs(c, b):
            off = c * CH
            return (pltpu.make_async_copy(
                        node_hbm.at[nidx_v.at[pl.ds(off, CH)]],
                        nbuf.at[b], gsem[b]),
                    pltpu.make_async_copy(
                        edge_hbm.at[eidx_v.at[pl.ds(off, CH)]],
                        ebuf.at[b], esem[b]))

        def w_descs(c, b):
            off = c * CH
            return (pltpu.make_async_copy(
                        nbuf.at[b], ng_out.at[pl.ds(base + off, CH)], wsem[b]),
                    pltpu.make_async_copy(
                        ebuf.at[b], eg_out.at[pl.ds(base + off, CH)], wsem[b]))

        def start(descs):
            for d in descs:
                d.start()

        def wait(descs):
            for d in descs:
                d.wait()

        for b in range(4):
            start(g_descs(b, b))

        def body(i, carry):
            for b in range(4):
                c = 4 * i + b
                wait(g_descs(c, b))
                start(w_descs(c, b))
            for b in range(4):
                c = 4 * i + b
                wait(w_descs(c, b))
                start(g_descs(c + 4, b))
            return carry

        lax.fori_loop(0, NCH // 4 - 1, body, 0)
        for b in range(4):
            c = NCH - 4 + b
            wait(g_descs(c, b))
            start(w_descs(c, b))
        for b in range(4):
            wait(w_descs(NCH - 4 + b, b))

        sbase = wid * SRC_RPW
        pltpu.sync_copy(sidx_hbm.at[pl.ds(sbase, SRC_RPW)],
                        nidx_v.at[pl.ds(0, SRC_RPW)])

        def sbody(c, carry):
            off = c * SCH
            pltpu.async_copy(
                node_hbm.at[nidx_v.at[pl.ds(off, SCH)]], sbuf, nsem).wait()
            pltpu.sync_copy(sbuf, src_out.at[pl.ds(sbase + off, SCH)])
            return carry

        lax.fori_loop(0, SNCH, sbody, 0)

    return k(node_tab, edge_tab, nbr_idx, eidx, sidx)


def _tc_body(ng_ref, eg_ref, src_ref, ts_ref, et_ref, nbr_ref,
             tw_ref, tb_ref, wq1_ref, wqt_ref,
             wk1_ref, wkt_ref, wke_ref, wv1_ref, wvt_ref, wve_ref,
             wo_ref, f1a_ref, f1b_ref, f1b_b_ref, f2_ref, f2b_ref,
             out_ref, gk_ref, gv_ref, wh_ref):
    f32 = jnp.float32
    bf = lambda x: x.astype(jnp.bfloat16)
    mm = lambda a, b: jax.lax.dot_general(
        a, b, (((1,), (0,)), ((), ())), preferred_element_type=f32)

    @pl.when(pl.program_id(0) == 0)
    def _():
        s = tw_ref[...][:, J0:] * f32(1000.0)
        cb = jnp.cos(tb_ref[...][:, J0:])
        sb = jnp.sin(tb_ref[...][:, J0:])
        rows = []
        spow = jnp.ones_like(s)
        fact = 1.0
        for p in range(P_DEG + 1):
            if p > 0:
                spow = spow * s
                fact = fact * p
            t = (cb, -sb, -cb, sb)[p % 4]
            rows.append(spow * (t * f32(1.0 / fact)))
        rows += [jnp.zeros_like(s)] * (P_PAD - P_DEG - 1)
        a_mat = jnp.concatenate(rows, axis=0)
        gk_ref[...] = bf(mm(a_mat, wkt_ref[...][J0:, :]))
        gv_ref[...] = bf(mm(a_mat, wvt_ref[...][J0:, :]))
        r = lax.broadcasted_iota(jnp.int32, (D_MODEL, D_MODEL), 0)
        c = lax.broadcasted_iota(jnp.int32, (D_MODEL, D_MODEL), 1)
        wh_ref[...] = bf(jnp.where((r // 64) == (c // 64), f32(0.125),
                                   f32(0.0)))

    delta = ts_ref[...] - et_ref[...]
    d3 = delta[:, :, None]
    arg_hi = (d3 * tw_ref[...][None, :, :J0]
              + tb_ref[...][None, :, :J0])
    te_hi = _fast_cos(arg_hi).reshape(BQ * L_SLOTS, J0)

    u = d3 * f32(0.001)
    ones = jnp.ones_like(u)
    p2 = jnp.concatenate([ones, u], axis=-1)
    u2 = u * u
    p4 = jnp.concatenate([p2, p2 * u2], axis=-1)
    u4 = u2 * u2
    p8 = jnp.concatenate([p4, p4 * u4], axis=-1)
    u8 = u4 * u4
    p8 = p8.reshape(BQ * L_SLOTS, 8)
    p8h = (p8.reshape(BQ, L_SLOTS, 8) * u8).reshape(BQ * L_SLOTS, 8)

    ng = ng_ref[...]
    eg = bf(eg_ref[...])
    te_hib = bf(te_hi)
    wkt_hib = bf(wkt_ref[...][:J0, :])
    wvt_hib = bf(wvt_ref[...][:J0, :])
    p8b, p8hb = bf(p8), bf(p8h)

    kk = (mm(ng, wk1_ref[...]) + mm(te_hib, wkt_hib)
          + mm(p8b, gk_ref[...][:8, :]) + mm(p8hb, gk_ref[...][8:, :])
          + mm(eg, wke_ref[...]))
    vv = (mm(ng, wv1_ref[...]) + mm(te_hib, wvt_hib)
          + mm(p8b, gv_ref[...][:8, :]) + mm(p8hb, gv_ref[...][8:, :])
          + mm(eg, wve_ref[...]))
    q = (mm(src_ref[...], wq1_ref[...])
         + mm(jnp.cos(tb_ref[...]), wqt_ref[...]))

    kk3 = kk.reshape(BQ, L_SLOTS, D_MODEL)
    vv3 = vv.reshape(BQ, L_SLOTS, D_MODEL)
    s_elem = q.reshape(BQ, 1, D_MODEL) * kk3
    s_exp = mm(bf(s_elem.reshape(BQ * L_SLOTS, D_MODEL)), wh_ref[...])
    s_exp = s_exp.reshape(BQ, L_SLOTS, D_MODEL)
    maskf = (nbr_ref[...] == 0).astype(f32)
    s_m = jnp.where(maskf[:, :, None] != 0, f32(-1e10), s_exp)
    m = jnp.max(s_m, axis=1, keepdims=True)
    e = jnp.exp(s_m - m)
    attn = e / jnp.sum(e, axis=1, keepdims=True)
    ctx = jnp.sum(attn * vv3, axis=1)

    ctxo = mm(ctx, wo_ref[...])
    h = jnp.maximum(
        mm(ctxo, f1a_ref[...]) + mm(src_ref[...], f1b_ref[...])
        + f1b_b_ref[...], f32(0.0))
    out_ref[...] = mm(h, f2_ref[...]) + f2b_ref[...]


def _tc_compute(ng, eg, src, ts, et, nbr, tw, tb, wq1, wqt,
                wk1, wkt, wke, wv1, wvt, wve, wo, f1a, f1b, f1bb, f2, f2b):
    grid = (BP // BQ,)
    def blk(shape):
        return pl.BlockSpec(shape, lambda g: (0,) * len(shape))
    in_specs = [
        pl.BlockSpec((BQ * L_SLOTS, D_FEAT), lambda g: (g, 0)),
        pl.BlockSpec((BQ * L_SLOTS, D_EDGE), lambda g: (g, 0)),
        pl.BlockSpec((BQ, D_FEAT), lambda g: (g, 0)),
        pl.BlockSpec((BQ, 1), lambda g: (g, 0)),
        pl.BlockSpec((BQ, L_SLOTS), lambda g: (g, 0)),
        pl.BlockSpec((BQ, L_SLOTS), lambda g: (g, 0)),
        blk((1, D_TIME)), blk((1, D_TIME)),
        blk((D_FEAT, D_MODEL)), blk((D_TIME, D_MODEL)),
        blk((D_FEAT, D_MODEL)), blk((D_TIME, D_MODEL)), blk((D_EDGE, D_MODEL)),
        blk((D_FEAT, D_MODEL)), blk((D_TIME, D_MODEL)), blk((D_EDGE, D_MODEL)),
        blk((D_MODEL, D_MODEL)),
        blk((D_MODEL, D_MODEL)), blk((D_FEAT, D_MODEL)), blk((1, D_MODEL)),
        blk((D_MODEL, D_FEAT)), blk((1, D_FEAT)),
    ]
    return pl.pallas_call(
        _tc_body,
        grid=grid,
        in_specs=in_specs,
        out_specs=pl.BlockSpec((BQ, D_FEAT), lambda g: (g, 0)),
        out_shape=jax.ShapeDtypeStruct((BP, D_FEAT), jnp.float32),
        scratch_shapes=[pltpu.VMEM((P_PAD, D_MODEL), jnp.bfloat16),
                        pltpu.VMEM((P_PAD, D_MODEL), jnp.bfloat16),
                        pltpu.VMEM((D_MODEL, D_MODEL), jnp.bfloat16)],
    )(ng, eg, src, ts, et, nbr, tw, tb, wq1, wqt,
      wk1, wkt, wke, wv1, wvt, wve, wo, f1a, f1b, f1bb, f2, f2b)


def kernel(node_features, edge_features, source_nodes, timestamps, neighbors,
           edge_idxs, edge_times, n_layers, n_neighbors, time_w, time_b,
           Wq, Wk, Wv, Wo, fc1_w, fc1_b, fc2_w, fc2_b):
    B, K, NN = neighbors.shape
    L = K * NN
    pad_rows = ROWS - B * L
    nbr_idx = jnp.pad(neighbors.reshape(-1), (0, pad_rows))
    eidx = jnp.pad(edge_idxs.reshape(-1), (0, pad_rows))
    sidx = jnp.pad(source_nodes, (0, BP - B))

    node_bf = node_features.astype(jnp.bfloat16)
    ng, eg, src = _sc_gather(node_bf, edge_features, nbr_idx, eidx, sidx)

    ts = jnp.pad(timestamps, (0, BP - B)).reshape(BP, 1)
    et = jnp.pad(edge_times.reshape(B, L), ((0, BP - B), (0, 0)))
    nbr = jnp.pad(neighbors.reshape(B, L), ((0, BP - B), (0, 0)))

    bf16 = jnp.bfloat16
    tw = time_w.reshape(1, D_TIME)
    tb = time_b.reshape(1, D_TIME)
    wq1, wqt = Wq[:D_FEAT].astype(bf16), Wq[D_FEAT:D_FEAT + D_TIME]
    wk1, wkt, wke = (Wk[:D_FEAT].astype(bf16), Wk[D_FEAT:D_FEAT + D_TIME],
                     Wk[D_FEAT + D_TIME:].astype(bf16))
    wv1, wvt, wve = (Wv[:D_FEAT].astype(bf16), Wv[D_FEAT:D_FEAT + D_TIME],
                     Wv[D_FEAT + D_TIME:].astype(bf16))
    f1a, f1b = fc1_w[:D_MODEL], fc1_w[D_MODEL:].astype(bf16)
    f1bb = fc1_b.reshape(1, D_MODEL)
    f2b = fc2_b.reshape(1, D_FEAT)

    out = _tc_compute(ng, eg,
                      src, ts, et, nbr, tw, tb, wq1, wqt,
                      wk1, wkt, wke, wv1, wvt, wve, Wo, f1a, f1b, f1bb,
                      fc2_w, f2b)
    return out[:B]


# speedup vs baseline: 1.0043x; 1.0043x over previous
import functools

import jax
import jax.numpy as jnp
from jax import lax
from jax.experimental import pallas as pl
from jax.experimental.pallas import tpu as pltpu
from jax.experimental.pallas import tpu_sc as plsc

D_FEAT = 128
D_EDGE = 16
D_TIME = 100
D_MODEL = 128
L_SLOTS = 40
BP = 10240
ROWS = BP * L_SLOTS
NW = 32
RPW = ROWS // NW
CH = 128
NCH = RPW // CH
SRC_RPW = BP // NW
SCH = 64
SNCH = SRC_RPW // SCH

BQ = 256
J0 = 32
JLO = D_TIME - J0
P_DEG = 12
P_PAD = 16

_INV2PI = 0.15915494309189535
_PI2_HI = 6.28125
_PI2_LO = 0.0019353071795864769


def _fast_cos(x):
    f32 = jnp.float32
    xa = jnp.abs(x)
    kf = (xa * f32(_INV2PI) + f32(0.5)).astype(jnp.int32).astype(f32)
    r = (xa - kf * f32(_PI2_HI)) - kf * f32(_PI2_LO)
    r2 = r * r
    acc = f32(4.7794773e-14)
    for c in (-1.1470746e-11, 2.0876757e-09, -2.7557319e-07,
              2.4801587e-05, -1.3888889e-03, 4.1666668e-02, -0.5):
        acc = acc * r2 + f32(c)
    return acc * r2 + f32(1.0)


def _sc_gather(node_tab, edge_tab, nbr_idx, eidx, sidx):
    mesh = plsc.VectorSubcoreMesh(core_axis_name="c", subcore_axis_name="s")

    @functools.partial(
        pl.kernel,
        out_type=[
            jax.ShapeDtypeStruct((ROWS, D_FEAT), jnp.bfloat16),
            jax.ShapeDtypeStruct((ROWS, D_EDGE), jnp.float32),
            jax.ShapeDtypeStruct((BP, D_FEAT), jnp.bfloat16),
        ],
        mesh=mesh,
        compiler_params=pltpu.CompilerParams(use_tc_tiling_on_sc=False),
        scratch_types=[
            pltpu.VMEM((RPW,), jnp.int32),
            pltpu.VMEM((RPW,), jnp.int32),
            pltpu.VMEM((4, CH, D_FEAT), jnp.bfloat16),
            pltpu.VMEM((4, CH, D_EDGE), jnp.float32),
            pltpu.VMEM((SCH, D_FEAT), jnp.bfloat16),
            pltpu.SemaphoreType.DMA,
        ] + [pltpu.SemaphoreType.DMA] * 12,
    )
    def k(node_hbm, edge_hbm, nidx_hbm, eidx_hbm, sidx_hbm,
          ng_out, eg_out, src_out,
          nidx_v, eidx_v, nbuf, ebuf, sbuf, nsem, *sems):
        gsem = sems[0:4]
        esem = sems[4:8]
        wsem = sems[8:12]
        wid = lax.axis_index("s") * 2 + lax.axis_index("c")
        base = wid * RPW
        pltpu.sync_copy(nidx_hbm.at[pl.ds(base, RPW)], nidx_v)
        pltpu.sync_copy(eidx_hbm.at[pl.ds(base, RPW)], eidx_v)

        def g_descs(c, b):
            off = c * CH
            return (pltpu.make_async_copy(
                        node_hbm.at[nidx_v.at[pl.ds(off, CH)]],
                        nbuf.at[b], gsem[b]),
                    pltpu.make_async_copy(
                        edge_hbm.at[eidx_v.at[pl.ds(off, CH)]],
                        ebuf.at[b], esem[b]))

        def w_descs(c, b):
            off = c * CH
            return (pltpu.make_async_copy(
                        nbuf.at[b], ng_out.at[pl.ds(base + off, CH)], wsem[b]),
                    pltpu.make_async_copy(
                        ebuf.at[b], eg_out.at[pl.ds(base + off, CH)], wsem[b]))

        def start(descs):
            for d in descs:
                d.start()

        def wait(descs):
            for d in descs:
                d.wait()

        for b in range(4):
            start(g_descs(b, b))

        def body(i, carry):
            for b in range(4):
                c = 4 * i + b
                wait(g_descs(c, b))
                start(w_descs(c, b))
            for b in range(4):
                c = 4 * i + b
                wait(w_descs(c, b))
                start(g_descs(c + 4, b))
            return carry

        lax.fori_loop(0, NCH // 4 - 1, body, 0)
        for b in range(4):
            c = NCH - 4 + b
            wait(g_descs(c, b))
            start(w_descs(c, b))
        for b in range(4):
            wait(w_descs(NCH - 4 + b, b))

        sbase = wid * SRC_RPW
        pltpu.sync_copy(sidx_hbm.at[pl.ds(sbase, SRC_RPW)],
                        nidx_v.at[pl.ds(0, SRC_RPW)])

        def sbody(c, carry):
            off = c * SCH
            pltpu.async_copy(
                node_hbm.at[nidx_v.at[pl.ds(off, SCH)]], sbuf, nsem).wait()
            pltpu.sync_copy(sbuf, src_out.at[pl.ds(sbase + off, SCH)])
            return carry

        lax.fori_loop(0, SNCH, sbody, 0)

    return k(node_tab, edge_tab, nbr_idx, eidx, sidx)


def _tc_body(ng_ref, eg_ref, src_ref, ts_ref, et_ref, nbr_ref,
             tw_ref, tb_ref, wq1_ref, wqt_ref,
             wk1_ref, wkt_ref, wke_ref, wv1_ref, wvt_ref, wve_ref,
             wo_ref, f1a_ref, f1b_ref, f1b_b_ref, f2_ref, f2b_ref,
             out_ref, gk_ref, gv_ref, wh_ref):
    f32 = jnp.float32
    bf = lambda x: x.astype(jnp.bfloat16)
    mm = lambda a, b: jax.lax.dot_general(
        a, b, (((1,), (0,)), ((), ())), preferred_element_type=f32)

    @pl.when(pl.program_id(0) == 0)
    def _():
        s = tw_ref[...][:, J0:] * f32(1000.0)
        cb = jnp.cos(tb_ref[...][:, J0:])
        sb = jnp.sin(tb_ref[...][:, J0:])
        rows = []
        spow = jnp.ones_like(s)
        fact = 1.0
        for p in range(P_DEG + 1):
            if p > 0:
                spow = spow * s
                fact = fact * p
            t = (cb, -sb, -cb, sb)[p % 4]
            rows.append(spow * (t * f32(1.0 / fact)))
        rows += [jnp.zeros_like(s)] * (P_PAD - P_DEG - 1)
        a_mat = jnp.concatenate(rows, axis=0)
        gk_ref[...] = mm(a_mat, wkt_ref[...][J0:, :])
        gv_ref[...] = mm(a_mat, wvt_ref[...][J0:, :])
        r = lax.broadcasted_iota(jnp.int32, (D_MODEL, D_MODEL), 0)
        c = lax.broadcasted_iota(jnp.int32, (D_MODEL, D_MODEL), 1)
        wh_ref[...] = bf(jnp.where((r // 64) == (c // 64), f32(0.125),
                                   f32(0.0)))

    delta = ts_ref[...] - et_ref[...]
    d3 = delta[:, :, None]
    arg_hi = (d3 * tw_ref[...][None, :, :J0]
              + tb_ref[...][None, :, :J0])
    te_hi = _fast_cos(arg_hi).reshape(BQ * L_SLOTS, J0)

    u = d3 * f32(0.001)
    ones = jnp.ones_like(u)
    p2 = jnp.concatenate([ones, u], axis=-1)
    u2 = u * u
    p4 = jnp.concatenate([p2, p2 * u2], axis=-1)
    u4 = u2 * u2
    p8 = jnp.concatenate([p4, p4 * u4], axis=-1)
    u8 = u4 * u4
    p8 = p8.reshape(BQ * L_SLOTS, 8)
    p8h = (p8.reshape(BQ, L_SLOTS, 8) * u8).reshape(BQ * L_SLOTS, 8)

    ng = ng_ref[...]
    eg = eg_ref[...]

    kk = (mm(ng, wk1_ref[...]) + mm(te_hi, wkt_ref[...][:J0, :])
          + mm(p8, gk_ref[...][:8, :]) + mm(p8h, gk_ref[...][8:, :])
          + mm(eg, wke_ref[...]))
    vv = (mm(ng, wv1_ref[...]) + mm(te_hi, wvt_ref[...][:J0, :])
          + mm(p8, gv_ref[...][:8, :]) + mm(p8h, gv_ref[...][8:, :])
          + mm(eg, wve_ref[...]))
    q = (mm(src_ref[...], wq1_ref[...])
         + mm(jnp.cos(tb_ref[...]), wqt_ref[...]))

    kk3 = kk.reshape(BQ, L_SLOTS, D_MODEL)
    vv3 = vv.reshape(BQ, L_SLOTS, D_MODEL)
    s_elem = q.reshape(BQ, 1, D_MODEL) * kk3
    s_exp = mm(bf(s_elem.reshape(BQ * L_SLOTS, D_MODEL)), wh_ref[...])
    s_exp = s_exp.reshape(BQ, L_SLOTS, D_MODEL)
    maskf = (nbr_ref[...] == 0).astype(f32)
    s_m = jnp.where(maskf[:, :, None] != 0, f32(-1e10), s_exp)
    m = jnp.max(s_m, axis=1, keepdims=True)
    e = jnp.exp(s_m - m)
    attn = e / jnp.sum(e, axis=1, keepdims=True)
    ctx = jnp.sum(attn * vv3, axis=1)

    ctxo = mm(ctx, wo_ref[...])
    h = jnp.maximum(
        mm(ctxo, f1a_ref[...]) + mm(src_ref[...], f1b_ref[...])
        + f1b_b_ref[...], f32(0.0))
    out_ref[...] = mm(h, f2_ref[...]) + f2b_ref[...]


def _tc_compute(ng, eg, src, ts, et, nbr, tw, tb, wq1, wqt,
                wk1, wkt, wke, wv1, wvt, wve, wo, f1a, f1b, f1bb, f2, f2b):
    grid = (BP // BQ,)
    def blk(shape):
        return pl.BlockSpec(shape, lambda g: (0,) * len(shape))
    in_specs = [
        pl.BlockSpec((BQ * L_SLOTS, D_FEAT), lambda g: (g, 0)),
        pl.BlockSpec((BQ * L_SLOTS, D_EDGE), lambda g: (g, 0)),
        pl.BlockSpec((BQ, D_FEAT), lambda g: (g, 0)),
        pl.BlockSpec((BQ, 1), lambda g: (g, 0)),
        pl.BlockSpec((BQ, L_SLOTS), lambda g: (g, 0)),
        pl.BlockSpec((BQ, L_SLOTS), lambda g: (g, 0)),
        blk((1, D_TIME)), blk((1, D_TIME)),
        blk((D_FEAT, D_MODEL)), blk((D_TIME, D_MODEL)),
        blk((D_FEAT, D_MODEL)), blk((D_TIME, D_MODEL)), blk((D_EDGE, D_MODEL)),
        blk((D_FEAT, D_MODEL)), blk((D_TIME, D_MODEL)), blk((D_EDGE, D_MODEL)),
        blk((D_MODEL, D_MODEL)),
        blk((D_MODEL, D_MODEL)), blk((D_FEAT, D_MODEL)), blk((1, D_MODEL)),
        blk((D_MODEL, D_FEAT)), blk((1, D_FEAT)),
    ]
    return pl.pallas_call(
        _tc_body,
        grid=grid,
        in_specs=in_specs,
        out_specs=pl.BlockSpec((BQ, D_FEAT), lambda g: (g, 0)),
        out_shape=jax.ShapeDtypeStruct((BP, D_FEAT), jnp.float32),
        scratch_shapes=[pltpu.VMEM((P_PAD, D_MODEL), jnp.float32),
                        pltpu.VMEM((P_PAD, D_MODEL), jnp.float32),
                        pltpu.VMEM((D_MODEL, D_MODEL), jnp.bfloat16)],
    )(ng, eg, src, ts, et, nbr, tw, tb, wq1, wqt,
      wk1, wkt, wke, wv1, wvt, wve, wo, f1a, f1b, f1bb, f2, f2b)


def kernel(node_features, edge_features, source_nodes, timestamps, neighbors,
           edge_idxs, edge_times, n_layers, n_neighbors, time_w, time_b,
           Wq, Wk, Wv, Wo, fc1_w, fc1_b, fc2_w, fc2_b):
    B, K, NN = neighbors.shape
    L = K * NN
    pad_rows = ROWS - B * L
    nbr_idx = jnp.pad(neighbors.reshape(-1), (0, pad_rows))
    eidx = jnp.pad(edge_idxs.reshape(-1), (0, pad_rows))
    sidx = jnp.pad(source_nodes, (0, BP - B))

    node_bf = node_features.astype(jnp.bfloat16)
    ng, eg, src = _sc_gather(node_bf, edge_features, nbr_idx, eidx, sidx)

    ts = jnp.pad(timestamps, (0, BP - B)).reshape(BP, 1)
    et = jnp.pad(edge_times.reshape(B, L), ((0, BP - B), (0, 0)))
    nbr = jnp.pad(neighbors.reshape(B, L), ((0, BP - B), (0, 0)))

    bf16 = jnp.bfloat16
    tw = time_w.reshape(1, D_TIME)
    tb = time_b.reshape(1, D_TIME)
    wq1, wqt = Wq[:D_FEAT].astype(bf16), Wq[D_FEAT:D_FEAT + D_TIME]
    wk1, wkt, wke = (Wk[:D_FEAT].astype(bf16), Wk[D_FEAT:D_FEAT + D_TIME],
                     Wk[D_FEAT + D_TIME:])
    wv1, wvt, wve = (Wv[:D_FEAT].astype(bf16), Wv[D_FEAT:D_FEAT + D_TIME],
                     Wv[D_FEAT + D_TIME:])
    f1a, f1b = fc1_w[:D_MODEL], fc1_w[D_MODEL:].astype(bf16)
    f1bb = fc1_b.reshape(1, D_MODEL)
    f2b = fc2_b.reshape(1, D_FEAT)

    out = _tc_compute(ng, eg,
                      src, ts, et, nbr, tw, tb, wq1, wqt,
                      wk1, wkt, wke, wv1, wvt, wve, Wo, f1a, f1b, f1bb,
                      fc2_w, f2b)
    return out[:B]


# speedup vs baseline: 1.0984x; 1.0937x over previous
import functools

import jax
import jax.numpy as jnp
from jax import lax
from jax.experimental import pallas as pl
from jax.experimental.pallas import tpu as pltpu
from jax.experimental.pallas import tpu_sc as plsc

D_FEAT = 128
D_EDGE = 16
D_TIME = 100
D_MODEL = 128
L_SLOTS = 40
BP = 10240
ROWS = BP * L_SLOTS
NW = 32
NSPLIT = 2
CH = 128
SCH = 32

BQ = 256
J0 = 32
JLO = D_TIME - J0
P_DEG = 12
P_PAD = 16

_INV2PI = 0.15915494309189535
_PI2_HI = 6.28125
_PI2_LO = 0.0019353071795864769


def _fast_cos(x):
    f32 = jnp.float32
    xa = jnp.abs(x)
    kf = (xa * f32(_INV2PI) + f32(0.5)).astype(jnp.int32).astype(f32)
    r = (xa - kf * f32(_PI2_HI)) - kf * f32(_PI2_LO)
    r2 = r * r
    acc = f32(4.7794773e-14)
    for c in (-1.1470746e-11, 2.0876757e-09, -2.7557319e-07,
              2.4801587e-05, -1.3888889e-03, 4.1666668e-02, -0.5):
        acc = acc * r2 + f32(c)
    return acc * r2 + f32(1.0)


def _sc_gather(node_tab, edge_tab, nbr_idx, eidx, sidx):
    rows = nbr_idx.shape[0]
    bp = sidx.shape[0]
    rpw = rows // NW
    nch = rpw // CH
    src_rpw = bp // NW
    snch = src_rpw // SCH
    mesh = plsc.VectorSubcoreMesh(core_axis_name="c", subcore_axis_name="s")

    @functools.partial(
        pl.kernel,
        out_type=[
            jax.ShapeDtypeStruct((rows, D_FEAT), jnp.bfloat16),
            jax.ShapeDtypeStruct((rows, D_EDGE), jnp.float32),
            jax.ShapeDtypeStruct((bp, D_FEAT), jnp.bfloat16),
        ],
        mesh=mesh,
        compiler_params=pltpu.CompilerParams(use_tc_tiling_on_sc=False),
        scratch_types=[
            pltpu.VMEM((rpw,), jnp.int32),
            pltpu.VMEM((rpw,), jnp.int32),
            pltpu.VMEM((2, CH, D_FEAT), jnp.bfloat16),
            pltpu.VMEM((2, CH, D_EDGE), jnp.float32),
            pltpu.VMEM((SCH, D_FEAT), jnp.bfloat16),
            pltpu.SemaphoreType.DMA,
        ] + [pltpu.SemaphoreType.DMA] * 6,
    )
    def k(node_hbm, edge_hbm, nidx_hbm, eidx_hbm, sidx_hbm,
          ng_out, eg_out, src_out,
          nidx_v, eidx_v, nbuf, ebuf, sbuf, nsem, *sems):
        gsem = sems[0:2]
        esem = sems[2:4]
        wsem = sems[4:6]
        wid = lax.axis_index("s") * 2 + lax.axis_index("c")
        base = wid * rpw
        pltpu.sync_copy(nidx_hbm.at[pl.ds(base, rpw)], nidx_v)
        pltpu.sync_copy(eidx_hbm.at[pl.ds(base, rpw)], eidx_v)

        def g_descs(c, b):
            off = c * CH
            return (pltpu.make_async_copy(
                        node_hbm.at[nidx_v.at[pl.ds(off, CH)]],
                        nbuf.at[b], gsem[b]),
                    pltpu.make_async_copy(
                        edge_hbm.at[eidx_v.at[pl.ds(off, CH)]],
                        ebuf.at[b], esem[b]))

        def w_descs(c, b):
            off = c * CH
            return (pltpu.make_async_copy(
                        nbuf.at[b], ng_out.at[pl.ds(base + off, CH)], wsem[b]),
                    pltpu.make_async_copy(
                        ebuf.at[b], eg_out.at[pl.ds(base + off, CH)], wsem[b]))

        def start(descs):
            for d in descs:
                d.start()

        def wait(descs):
            for d in descs:
                d.wait()

        for b in range(2):
            start(g_descs(b, b))

        def body(i, carry):
            for b in range(2):
                c = 2 * i + b
                wait(g_descs(c, b))
                start(w_descs(c, b))
            for b in range(2):
                c = 2 * i + b
                wait(w_descs(c, b))
                start(g_descs(c + 2, b))
            return carry

        lax.fori_loop(0, nch // 2 - 1, body, 0)
        for b in range(2):
            c = nch - 2 + b
            wait(g_descs(c, b))
            start(w_descs(c, b))
        for b in range(2):
            wait(w_descs(nch - 2 + b, b))

        sbase = wid * src_rpw
        pltpu.sync_copy(sidx_hbm.at[pl.ds(sbase, src_rpw)],
                        nidx_v.at[pl.ds(0, src_rpw)])

        def sbody(c, carry):
            off = c * SCH
            pltpu.async_copy(
                node_hbm.at[nidx_v.at[pl.ds(off, SCH)]], sbuf, nsem).wait()
            pltpu.sync_copy(sbuf, src_out.at[pl.ds(sbase + off, SCH)])
            return carry

        lax.fori_loop(0, snch, sbody, 0)

    return k(node_tab, edge_tab, nbr_idx, eidx, sidx)


def _tc_body(ng_ref, eg_ref, src_ref, ts_ref, et_ref, nbr_ref,
             tw_ref, tb_ref, wq1_ref, wqt_ref,
             wk1_ref, wkt_ref, wke_ref, wv1_ref, wvt_ref, wve_ref,
             wo_ref, f1a_ref, f1b_ref, f1b_b_ref, f2_ref, f2b_ref,
             out_ref, gk_ref, gv_ref, wh_ref):
    f32 = jnp.float32
    bf = lambda x: x.astype(jnp.bfloat16)
    mm = lambda a, b: jax.lax.dot_general(
        a, b, (((1,), (0,)), ((), ())), preferred_element_type=f32)

    @pl.when(pl.program_id(0) == 0)
    def _():
        s = tw_ref[...][:, J0:] * f32(1000.0)
        cb = jnp.cos(tb_ref[...][:, J0:])
        sb = jnp.sin(tb_ref[...][:, J0:])
        rows = []
        spow = jnp.ones_like(s)
        fact = 1.0
        for p in range(P_DEG + 1):
            if p > 0:
                spow = spow * s
                fact = fact * p
            t = (cb, -sb, -cb, sb)[p % 4]
            rows.append(spow * (t * f32(1.0 / fact)))
        rows += [jnp.zeros_like(s)] * (P_PAD - P_DEG - 1)
        a_mat = jnp.concatenate(rows, axis=0)
        gk_ref[...] = mm(a_mat, wkt_ref[...][J0:, :])
        gv_ref[...] = mm(a_mat, wvt_ref[...][J0:, :])
        r = lax.broadcasted_iota(jnp.int32, (D_MODEL, D_MODEL), 0)
        c = lax.broadcasted_iota(jnp.int32, (D_MODEL, D_MODEL), 1)
        wh_ref[...] = bf(jnp.where((r // 64) == (c // 64), f32(0.125),
                                   f32(0.0)))

    delta = ts_ref[...] - et_ref[...]
    d3 = delta[:, :, None]
    arg_hi = (d3 * tw_ref[...][None, :, :J0]
              + tb_ref[...][None, :, :J0])
    te_hi = _fast_cos(arg_hi).reshape(BQ * L_SLOTS, J0)

    u = d3 * f32(0.001)
    ones = jnp.ones_like(u)
    p2 = jnp.concatenate([ones, u], axis=-1)
    u2 = u * u
    p4 = jnp.concatenate([p2, p2 * u2], axis=-1)
    u4 = u2 * u2
    p8 = jnp.concatenate([p4, p4 * u4], axis=-1)
    u8 = u4 * u4
    p8 = p8.reshape(BQ * L_SLOTS, 8)
    p8h = (p8.reshape(BQ, L_SLOTS, 8) * u8).reshape(BQ * L_SLOTS, 8)

    ng = ng_ref[...]
    eg = eg_ref[...]

    kk = (mm(ng, wk1_ref[...]) + mm(te_hi, wkt_ref[...][:J0, :])
          + mm(p8, gk_ref[...][:8, :]) + mm(p8h, gk_ref[...][8:, :])
          + mm(eg, wke_ref[...]))
    vv = (mm(ng, wv1_ref[...]) + mm(te_hi, wvt_ref[...][:J0, :])
          + mm(p8, gv_ref[...][:8, :]) + mm(p8h, gv_ref[...][8:, :])
          + mm(eg, wve_ref[...]))
    q = (mm(src_ref[...], wq1_ref[...])
         + mm(jnp.cos(tb_ref[...]), wqt_ref[...]))

    kk3 = kk.reshape(BQ, L_SLOTS, D_MODEL)
    vv3 = vv.reshape(BQ, L_SLOTS, D_MODEL)
    s_elem = q.reshape(BQ, 1, D_MODEL) * kk3
    s_exp = mm(bf(s_elem.reshape(BQ * L_SLOTS, D_MODEL)), wh_ref[...])
    s_exp = s_exp.reshape(BQ, L_SLOTS, D_MODEL)
    maskf = (nbr_ref[...] == 0).astype(f32)
    s_m = jnp.where(maskf[:, :, None] != 0, f32(-1e10), s_exp)
    m = jnp.max(s_m, axis=1, keepdims=True)
    e = jnp.exp(s_m - m)
    attn = e / jnp.sum(e, axis=1, keepdims=True)
    ctx = jnp.sum(attn * vv3, axis=1)

    ctxo = mm(ctx, wo_ref[...])
    h = jnp.maximum(
        mm(ctxo, f1a_ref[...]) + mm(src_ref[...], f1b_ref[...])
        + f1b_b_ref[...], f32(0.0))
    out_ref[...] = mm(h, f2_ref[...]) + f2b_ref[...]


def _tc_compute(ng, eg, src, ts, et, nbr, tw, tb, wq1, wqt,
                wk1, wkt, wke, wv1, wvt, wve, wo, f1a, f1b, f1bb, f2, f2b):
    bp = src.shape[0]
    grid = (bp // BQ,)
    def blk(shape):
        return pl.BlockSpec(shape, lambda g: (0,) * len(shape))
    in_specs = [
        pl.BlockSpec((BQ * L_SLOTS, D_FEAT), lambda g: (g, 0)),
        pl.BlockSpec((BQ * L_SLOTS, D_EDGE), lambda g: (g, 0)),
        pl.BlockSpec((BQ, D_FEAT), lambda g: (g, 0)),
        pl.BlockSpec((BQ, 1), lambda g: (g, 0)),
        pl.BlockSpec((BQ, L_SLOTS), lambda g: (g, 0)),
        pl.BlockSpec((BQ, L_SLOTS), lambda g: (g, 0)),
        blk((1, D_TIME)), blk((1, D_TIME)),
        blk((D_FEAT, D_MODEL)), blk((D_TIME, D_MODEL)),
        blk((D_FEAT, D_MODEL)), blk((D_TIME, D_MODEL)), blk((D_EDGE, D_MODEL)),
        blk((D_FEAT, D_MODEL)), blk((D_TIME, D_MODEL)), blk((D_EDGE, D_MODEL)),
        blk((D_MODEL, D_MODEL)),
        blk((D_MODEL, D_MODEL)), blk((D_FEAT, D_MODEL)), blk((1, D_MODEL)),
        blk((D_MODEL, D_FEAT)), blk((1, D_FEAT)),
    ]
    return pl.pallas_call(
        _tc_body,
        grid=grid,
        in_specs=in_specs,
        out_specs=pl.BlockSpec((BQ, D_FEAT), lambda g: (g, 0)),
        out_shape=jax.ShapeDtypeStruct((bp, D_FEAT), jnp.float32),
        scratch_shapes=[pltpu.VMEM((P_PAD, D_MODEL), jnp.float32),
                        pltpu.VMEM((P_PAD, D_MODEL), jnp.float32),
                        pltpu.VMEM((D_MODEL, D_MODEL), jnp.bfloat16)],
    )(ng, eg, src, ts, et, nbr, tw, tb, wq1, wqt,
      wk1, wkt, wke, wv1, wvt, wve, wo, f1a, f1b, f1bb, f2, f2b)


def kernel(node_features, edge_features, source_nodes, timestamps, neighbors,
           edge_idxs, edge_times, n_layers, n_neighbors, time_w, time_b,
           Wq, Wk, Wv, Wo, fc1_w, fc1_b, fc2_w, fc2_b):
    B, K, NN = neighbors.shape
    L = K * NN
    pad_rows = ROWS - B * L
    nbr_idx = jnp.pad(neighbors.reshape(-1), (0, pad_rows))
    eidx = jnp.pad(edge_idxs.reshape(-1), (0, pad_rows))
    sidx = jnp.pad(source_nodes, (0, BP - B))

    node_bf = node_features.astype(jnp.bfloat16)

    ts = jnp.pad(timestamps, (0, BP - B)).reshape(BP, 1)
    et = jnp.pad(edge_times.reshape(B, L), ((0, BP - B), (0, 0)))
    nbr = jnp.pad(neighbors.reshape(B, L), ((0, BP - B), (0, 0)))

    bf16 = jnp.bfloat16
    tw = time_w.reshape(1, D_TIME)
    tb = time_b.reshape(1, D_TIME)
    wq1, wqt = Wq[:D_FEAT].astype(bf16), Wq[D_FEAT:D_FEAT + D_TIME]
    wk1, wkt, wke = (Wk[:D_FEAT].astype(bf16), Wk[D_FEAT:D_FEAT + D_TIME],
                     Wk[D_FEAT + D_TIME:])
    wv1, wvt, wve = (Wv[:D_FEAT].astype(bf16), Wv[D_FEAT:D_FEAT + D_TIME],
                     Wv[D_FEAT + D_TIME:])
    f1a, f1b = fc1_w[:D_MODEL], fc1_w[D_MODEL:].astype(bf16)
    f1bb = fc1_b.reshape(1, D_MODEL)
    f2b = fc2_b.reshape(1, D_FEAT)

    bph = BP // NSPLIT
    rh = ROWS // NSPLIT
    outs = []
    for h in range(NSPLIT):
        qs, rs = h * bph, h * rh
        ng, eg, src = _sc_gather(node_bf, edge_features,
                                 nbr_idx[rs:rs + rh], eidx[rs:rs + rh],
                                 sidx[qs:qs + bph])
        outs.append(_tc_compute(
            ng, eg, src, ts[qs:qs + bph], et[qs:qs + bph],
            nbr[qs:qs + bph], tw, tb, wq1, wqt,
            wk1, wkt, wke, wv1, wvt, wve, Wo, f1a, f1b, f1bb,
            fc2_w, f2b))
    return jnp.concatenate(outs, axis=0)[:B]
